# Initial kernel scaffold; baseline (speedup 1.0000x reference)
import jax
import jax.numpy as jnp
from jax.experimental import pallas as pl


def kernel(img_real, img_imag, trj):
    raise NotImplementedError("write your pallas kernel here")


# speedup vs baseline: 1.8494x; 1.8494x over previous
import functools

import jax
import jax.numpy as jnp
from jax.experimental import pallas as pl

_IM = 64
_NC = 2
_NT = 16384
_TB = 2048


def _body(trj_ref, a_ref, out_r_ref, out_i_ref):
    two_pi = 2.0 * jnp.pi
    k0 = (trj_ref[0:1, :] - 0.5) * float(_IM)
    k1 = (trj_ref[1:2, :] - 0.5) * float(_IM)
    x = jax.lax.broadcasted_iota(jnp.int32, (_IM, 1), 0).astype(jnp.float32)
    r = (x - float(_IM // 2)) / float(_IM)
    ang0 = (-two_pi) * (r * k0)
    ang1 = (-two_pi) * (r * k1)
    e0r = jnp.cos(ang0)
    e0i = jnp.sin(ang0)
    e1r = jnp.cos(ang1)
    e1i = jnp.sin(ang1)
    e1 = jnp.concatenate([e1r, e1i], axis=0)
    tmp = jnp.dot(a_ref[...], e1, preferred_element_type=jnp.float32)
    t0r = tmp[0:64]
    t0i = tmp[64:128]
    t1r = tmp[128:192]
    t1i = tmp[192:256]
    k0r = jnp.sum(e0r * t0r - e0i * t0i, axis=0, keepdims=True)
    k0i = jnp.sum(e0r * t0i + e0i * t0r, axis=0, keepdims=True)
    k1r = jnp.sum(e0r * t1r - e0i * t1i, axis=0, keepdims=True)
    k1i = jnp.sum(e0r * t1i + e0i * t1r, axis=0, keepdims=True)
    out_r_ref[...] = jnp.concatenate([k0r, k1r], axis=0)
    out_i_ref[...] = jnp.concatenate([k0i, k1i], axis=0)


@functools.partial(jax.jit, static_argnames=("interpret",))
def _nufft(img_real, img_imag, trj, interpret=False):
    ir = img_real[0]
    ii = img_imag[0]

    def coil_block(c):
        return jnp.concatenate(
            [
                jnp.concatenate([ir[c], -ii[c]], axis=1),
                jnp.concatenate([ii[c], ir[c]], axis=1),
            ],
            axis=0,
        )

    a = jnp.concatenate([coil_block(0), coil_block(1)], axis=0)
    trj_t = trj[0].T

    grid = (_NT // _TB,)
    out_r, out_i = pl.pallas_call(
        _body,
        grid=grid,
        in_specs=[
            pl.BlockSpec((2, _TB), lambda i: (0, i)),
            pl.BlockSpec((256, 128), lambda i: (0, 0)),
        ],
        out_specs=[
            pl.BlockSpec((_NC, _TB), lambda i: (0, i)),
            pl.BlockSpec((_NC, _TB), lambda i: (0, i)),
        ],
        out_shape=[
            jax.ShapeDtypeStruct((_NC, _NT), jnp.float32),
            jax.ShapeDtypeStruct((_NC, _NT), jnp.float32),
        ],
        interpret=interpret,
    )(trj_t, a)
    return (out_r + 1j * out_i).astype(jnp.complex64)[None]


def kernel(img_real, img_imag, trj):
    return _nufft(img_real, img_imag, trj)


# speedup vs baseline: 5.0706x; 2.7417x over previous
import functools

import jax
import jax.numpy as jnp
from jax.experimental import pallas as pl

_IM = 64
_NC = 2
_NT = 16384
_TB = 2048


def _build_exp(theta):
    wr = jnp.cos(theta)
    wi = jnp.sin(theta)
    pows = [(wr, wi)]
    for _ in range(5):
        pr, pi_ = pows[-1]
        pows.append((pr * pr - pi_ * pi_, 2.0 * pr * pi_))
    p32r, p32i = pows[5]
    er, ei = p32r, -p32i
    for s in range(6):
        pr, pi_ = pows[s]
        nr = er * pr - ei * pi_
        ni = er * pi_ + ei * pr
        er = jnp.concatenate([er, nr], axis=0)
        ei = jnp.concatenate([ei, ni], axis=0)
    return er, ei


def _body(trj_ref, a_ref, out_r_ref, out_i_ref):
    two_pi = 2.0 * jnp.pi
    th0 = (-two_pi) * (trj_ref[0:1, :] - 0.5)
    th1 = (-two_pi) * (trj_ref[1:2, :] - 0.5)
    e0r, e0i = _build_exp(th0)
    e1r, e1i = _build_exp(th1)
    e1 = jnp.concatenate([e1r, e1i], axis=0)
    tmp = jnp.dot(a_ref[...], e1, preferred_element_type=jnp.float32)
    t0r = tmp[0:64]
    t0i = tmp[64:128]
    t1r = tmp[128:192]
    t1i = tmp[192:256]
    k0r = jnp.sum(e0r * t0r - e0i * t0i, axis=0, keepdims=True)
    k0i = jnp.sum(e0r * t0i + e0i * t0r, axis=0, keepdims=True)
    k1r = jnp.sum(e0r * t1r - e0i * t1i, axis=0, keepdims=True)
    k1i = jnp.sum(e0r * t1i + e0i * t1r, axis=0, keepdims=True)
    out_r_ref[...] = jnp.concatenate([k0r, k1r], axis=0)
    out_i_ref[...] = jnp.concatenate([k0i, k1i], axis=0)


@functools.partial(jax.jit, static_argnames=("interpret",))
def _nufft(img_real, img_imag, trj, interpret=False):
    ir = img_real[0]
    ii = img_imag[0]

    def coil_block(c):
        return jnp.concatenate(
            [
                jnp.concatenate([ir[c], -ii[c]], axis=1),
                jnp.concatenate([ii[c], ir[c]], axis=1),
            ],
            axis=0,
        )

    a = jnp.concatenate([coil_block(0), coil_block(1)], axis=0)
    trj_t = trj[0].T

    grid = (_NT // _TB,)
    out_r, out_i = pl.pallas_call(
        _body,
        grid=grid,
        in_specs=[
            pl.BlockSpec((2, _TB), lambda i: (0, i)),
            pl.BlockSpec((256, 128), lambda i: (0, 0)),
        ],
        out_specs=[
            pl.BlockSpec((_NC, _TB), lambda i: (0, i)),
            pl.BlockSpec((_NC, _TB), lambda i: (0, i)),
        ],
        out_shape=[
            jax.ShapeDtypeStruct((_NC, _NT), jnp.float32),
            jax.ShapeDtypeStruct((_NC, _NT), jnp.float32),
        ],
        interpret=interpret,
    )(trj_t, a)
    return (out_r + 1j * out_i).astype(jnp.complex64)[None]


def kernel(img_real, img_imag, trj):
    return _nufft(img_real, img_imag, trj)
